# baseline (speedup 1.0000x reference)
import functools

import jax
import jax.numpy as jnp
from jax import lax
from jax.experimental import pallas as pl
from jax.experimental.pallas import tpu as pltpu
from jax.experimental.pallas import tpu_sc as plsc

_RES = 1000000
_LAT = 64
_N = 1048576
_NC = 2
_NS = 16
_NW = _NC * _NS
_BW = _N // _NW
_C = 128
_G = _BW // _C
_NSLOT = 3


def _body(u_hbm, table_hbm, out_hbm, *args):
    u_v = args[0:3]
    idx_a = args[3:6]
    idx_b = args[6:9]
    rows_a = args[9:12]
    rows_b = args[12:15]
    rows_o = args[15:18]
    sem_g = args[18:21]
    sem_o = args[21:24]
    wid = lax.axis_index("s") * _NC + lax.axis_index("c")
    base0 = wid * _BW
    scale = jnp.float32(_RES - 1)

    def gather_copies(b):
        return [
            pltpu.make_async_copy(
                table_hbm.at[idx_a[b].at[0]], rows_a[b], sem_g[b]),
            pltpu.make_async_copy(
                table_hbm.at[idx_b[b].at[0]], rows_b[b], sem_g[b]),
        ]

    def out_copy(g, b):
        return pltpu.make_async_copy(
            rows_o[b],
            out_hbm.at[pl.ds(
                pl.multiple_of(wid * (_BW // 2) + g * (_C // 2), _C // 2),
                _C // 2)],
            sem_o[b])

    def prep(g, b):
        base = pl.multiple_of(base0 + g * _C, _C)
        pltpu.sync_copy(u_hbm.at[pl.ds(base, _C)], u_v[b])

        def idx_body(k, c):
            off = k * 16
            u16 = u_v[b][pl.ds(off, 16)]
            f = u16 * scale
            ix = f.astype(jnp.int32)
            fl = ix.astype(jnp.float32)
            idx_a[b][0, pl.ds(off, 16)] = lax.shift_right_logical(ix, 1)
            idx_b[b][0, pl.ds(off, 16)] = lax.shift_right_logical(ix + 1, 1)
            u_v[b][pl.ds(off, 16)] = f - fl
            return c

        lax.fori_loop(0, _C // 16, idx_body, 0, unroll=True)
        for c in gather_copies(b):
            c.start()

    def cons(g, b, first):
        for c in gather_copies(b):
            c.wait()
        if not first:
            out_copy(g, b).wait()

        def lerp_body(blk, c):
            i0 = blk * 16
            al16 = u_v[b][pl.ds(i0, 16)]
            ia16 = idx_a[b][0, pl.ds(i0, 16)]
            ib16 = idx_b[b][0, pl.ds(i0, 16)]
            par16 = ib16 - ia16
            for l in range(16):
                i = i0 + l
                al = jnp.full((16,), al16[l], jnp.float32)
                acol = par16[l] * 64
                bcol = 64 - acol
                orow = blk * 8 + (l // 2)
                ocol = (l & 1) * 64
                for c4 in range(4):
                    a = rows_a[b][i, pl.ds(acol + c4 * 16, 16)]
                    bb = rows_b[b][i, pl.ds(bcol + c4 * 16, 16)]
                    rows_o[b][orow, pl.ds(ocol + c4 * 16, 16)] = (
                        a + al * (bb - a))
            return c

        lax.fori_loop(0, _C // 16, lerp_body, 0, unroll=False)
        out_copy(g, b).start()

    prep(0, 0)
    prep(1, 1)
    prep(2, 2)
    cons(0, 0, True)
    prep(3, 0)
    cons(1, 1, True)
    prep(4, 1)
    cons(2, 2, True)
    prep(5, 2)
    cons(3, 0, False)
    prep(6, 0)

    def triple(t, carry):
        for j in range(3):
            g = 4 + t * 3 + j
            b = (4 + j) % _NSLOT
            cons(g, b, False)

            @pl.when(g + 3 < _G)
            def _():
                prep(g + 3, b)
        return carry

    lax.fori_loop(0, (_G - 4) // 3, triple, 0, unroll=False)

    out_copy(_G - 3, (_G - 3) % _NSLOT).wait()
    out_copy(_G - 2, (_G - 2) % _NSLOT).wait()
    out_copy(_G - 1, (_G - 1) % _NSLOT).wait()


def kernel(uList, table):
    mesh = plsc.VectorSubcoreMesh(core_axis_name="c", subcore_axis_name="s")
    table2 = table.reshape(_RES // 2, 2 * _LAT)
    scr = (
        [pltpu.VMEM((_C,), jnp.float32) for _ in range(_NSLOT)]
        + [pltpu.VMEM((1, _C), jnp.int32) for _ in range(_NSLOT)]
        + [pltpu.VMEM((1, _C), jnp.int32) for _ in range(_NSLOT)]
        + [pltpu.VMEM((_C, 2 * _LAT), jnp.float32) for _ in range(_NSLOT)]
        + [pltpu.VMEM((_C, 2 * _LAT), jnp.float32) for _ in range(_NSLOT)]
        + [pltpu.VMEM((_C // 2, 2 * _LAT), jnp.float32) for _ in range(_NSLOT)]
        + [pltpu.SemaphoreType.DMA for _ in range(2 * _NSLOT)]
    )
    k = functools.partial(
        pl.kernel,
        mesh=mesh,
        out_type=jax.ShapeDtypeStruct((_N // 2, 2 * _LAT), jnp.float32),
        scratch_types=scr,
    )(_body)
    out2 = k(uList, table2)
    return out2.reshape(_N, _LAT)


# speedup vs baseline: 1.4004x; 1.4004x over previous
import functools

import jax
import jax.numpy as jnp
from jax import lax
from jax.experimental import pallas as pl
from jax.experimental.pallas import tpu as pltpu
from jax.experimental.pallas import tpu_sc as plsc

_RES = 1000000
_LAT = 64
_N = 1048576
_NC = 2
_NS = 16
_NW = _NC * _NS
_BW = _N // _NW
_C = 512
_G = _BW // _C
_SUB = _C // 128


def _body(u_hbm, table_hbm, out_hbm,
          u0, u1, ia0, ia1, ib0, ib1, rows_a, rows_b, sem_g, sem_o):
    u_v = (u0, u1)
    idx_a = (ia0, ia1)
    idx_b = (ib0, ib1)
    wid = lax.axis_index("s") * _NC + lax.axis_index("c")
    base0 = wid * _BW
    scale = jnp.float32(_RES - 1)

    def ga_copies(s):
        return [pltpu.make_async_copy(
            table_hbm.at[idx_a[s].at[j]],
            rows_a.at[pl.ds(j * 128, 128)], sem_g) for j in range(_SUB)]

    def gb_copies(s):
        return [pltpu.make_async_copy(
            table_hbm.at[idx_b[s].at[j]],
            rows_b.at[pl.ds(j * 128, 128)], sem_g) for j in range(_SUB)]

    def out_copy(g):
        return pltpu.make_async_copy(
            rows_b, out_hbm.at[pl.ds(base0 + g * _C, _C)], sem_o)

    def prep_meta(g, s):
        base = base0 + g * _C
        pltpu.sync_copy(u_hbm.at[pl.ds(base, _C)], u_v[s])

        def idx_body(j, c):
            for k in range(8):
                off = j * 128 + k * 16
                u16 = u_v[s][pl.ds(off, 16)]
                f = u16 * scale
                ix = f.astype(jnp.int32)
                fl = ix.astype(jnp.float32)
                idx_a[s][j, pl.ds(k * 16, 16)] = ix
                idx_b[s][j, pl.ds(k * 16, 16)] = ix + 1
                u_v[s][pl.ds(off, 16)] = f - fl
            return c

        lax.fori_loop(0, _SUB, idx_body, 0, unroll=True)

    def lerp(s):
        def lerp_body(blk, c):
            i0 = blk * 16
            al16 = u_v[s][pl.ds(i0, 16)]
            for l in range(16):
                al = jnp.full((16,), al16[l], jnp.float32)
                for r in range(4):
                    a = rows_a[i0 + l, pl.ds(r * 16, 16)]
                    bb = rows_b[i0 + l, pl.ds(r * 16, 16)]
                    rows_b[i0 + l, pl.ds(r * 16, 16)] = a + al * (bb - a)
            return c

        lax.fori_loop(0, _C // 16, lerp_body, 0, unroll=False)

    def chunk(g, s, last):
        if not last:
            prep_meta(g + 1, 1 - s)
        for c in ga_copies(s):
            c.wait()
        for c in gb_copies(s):
            c.wait()
        lerp(s)
        if not last:
            for c in ga_copies(1 - s):
                c.start()
        out_copy(g).start()
        out_copy(g).wait()
        if not last:
            for c in gb_copies(1 - s):
                c.start()

    prep_meta(0, 0)
    for c in ga_copies(0):
        c.start()
    for c in gb_copies(0):
        c.start()

    def pair(t, carry):
        chunk(2 * t, 0, False)
        chunk(2 * t + 1, 1, False)
        return carry

    lax.fori_loop(0, _G // 2 - 1, pair, 0, unroll=False)

    chunk(_G - 2, 0, False)
    chunk(_G - 1, 1, True)


def kernel(uList, table):
    mesh = plsc.VectorSubcoreMesh(core_axis_name="c", subcore_axis_name="s")
    k = functools.partial(
        pl.kernel,
        mesh=mesh,
        out_type=jax.ShapeDtypeStruct((_N, _LAT), jnp.float32),
        compiler_params=pltpu.CompilerParams(use_tc_tiling_on_sc=False),
        scratch_types=[
            pltpu.VMEM((_C,), jnp.float32),
            pltpu.VMEM((_C,), jnp.float32),
            pltpu.VMEM((_SUB, 128), jnp.int32),
            pltpu.VMEM((_SUB, 128), jnp.int32),
            pltpu.VMEM((_SUB, 128), jnp.int32),
            pltpu.VMEM((_SUB, 128), jnp.int32),
            pltpu.VMEM((_C, _LAT), jnp.float32),
            pltpu.VMEM((_C, _LAT), jnp.float32),
            pltpu.SemaphoreType.DMA,
            pltpu.SemaphoreType.DMA,
        ],
    )(_body)
    return k(uList, table)


# speedup vs baseline: 1.4077x; 1.0052x over previous
import functools

import jax
import jax.numpy as jnp
from jax import lax
from jax.experimental import pallas as pl
from jax.experimental.pallas import tpu as pltpu
from jax.experimental.pallas import tpu_sc as plsc

_RES = 1000000
_LAT = 64
_N = 1048576
_NC = 2
_NS = 16
_NW = _NC * _NS
_BW = _N // _NW
_C = 512
_G = _BW // _C
_SUB = _C // 128


def _body(u_hbm, table_hbm, out_hbm,
          u0, u1, ia0, ia1, ib0, ib1, rows_a, rows_b,
          sg0, sg1, sg2, sg3, sem_o):
    u_v = (u0, u1)
    idx_a = (ia0, ia1)
    idx_b = (ib0, ib1)
    sem_g = (sg0, sg1, sg2, sg3)
    wid = lax.axis_index("s") * _NC + lax.axis_index("c")
    base0 = wid * _BW
    scale = jnp.float32(_RES - 1)

    def ga_copy(s, j):
        return pltpu.make_async_copy(
            table_hbm.at[idx_a[s].at[j]],
            rows_a.at[pl.ds(j * 128, 128)], sem_g[j])

    def gb_copy(s, j):
        return pltpu.make_async_copy(
            table_hbm.at[idx_b[s].at[j]],
            rows_b.at[pl.ds(j * 128, 128)], sem_g[j])

    def out_copy(g, j):
        return pltpu.make_async_copy(
            rows_b.at[pl.ds(j * 128, 128)],
            out_hbm.at[pl.ds(base0 + g * _C + j * 128, 128)], sem_o)

    def prep_meta(g, s):
        base = base0 + g * _C
        pltpu.sync_copy(u_hbm.at[pl.ds(base, _C)], u_v[s])

        def idx_body(j, c):
            for k in range(8):
                off = j * 128 + k * 16
                u16 = u_v[s][pl.ds(off, 16)]
                f = u16 * scale
                ix = f.astype(jnp.int32)
                fl = ix.astype(jnp.float32)
                idx_a[s][j, pl.ds(k * 16, 16)] = ix
                idx_b[s][j, pl.ds(k * 16, 16)] = ix + 1
                u_v[s][pl.ds(off, 16)] = f - fl
            return c

        lax.fori_loop(0, _SUB, idx_body, 0, unroll=True)

    def lerp_sub(s, j):
        def lerp_body(blk, c):
            i0 = j * 128 + blk * 16
            al16 = u_v[s][pl.ds(i0, 16)]
            for l in range(16):
                al = jnp.full((16,), al16[l], jnp.float32)
                for r in range(4):
                    a = rows_a[i0 + l, pl.ds(r * 16, 16)]
                    bb = rows_b[i0 + l, pl.ds(r * 16, 16)]
                    rows_b[i0 + l, pl.ds(r * 16, 16)] = a + al * (bb - a)
            return c

        lax.fori_loop(0, 8, lerp_body, 0, unroll=False)

    def chunk(g, s, last):
        if not last:
            prep_meta(g + 1, 1 - s)
        for j in range(_SUB):
            ga_copy(s, j).wait()
            gb_copy(s, j).wait()
            lerp_sub(s, j)
            out_copy(g, j).start()
        if not last:
            for j in range(_SUB):
                ga_copy(1 - s, j).start()
        for j in range(_SUB):
            out_copy(g, j).wait()
        if not last:
            for j in range(_SUB):
                gb_copy(1 - s, j).start()

    prep_meta(0, 0)
    for j in range(_SUB):
        ga_copy(0, j).start()
        gb_copy(0, j).start()

    def pair(t, carry):
        chunk(2 * t, 0, False)
        chunk(2 * t + 1, 1, False)
        return carry

    lax.fori_loop(0, _G // 2 - 1, pair, 0, unroll=False)

    chunk(_G - 2, 0, False)
    chunk(_G - 1, 1, True)


def kernel(uList, table):
    mesh = plsc.VectorSubcoreMesh(core_axis_name="c", subcore_axis_name="s")
    k = functools.partial(
        pl.kernel,
        mesh=mesh,
        out_type=jax.ShapeDtypeStruct((_N, _LAT), jnp.float32),
        compiler_params=pltpu.CompilerParams(use_tc_tiling_on_sc=False),
        scratch_types=[
            pltpu.VMEM((_C,), jnp.float32),
            pltpu.VMEM((_C,), jnp.float32),
            pltpu.VMEM((_SUB, 128), jnp.int32),
            pltpu.VMEM((_SUB, 128), jnp.int32),
            pltpu.VMEM((_SUB, 128), jnp.int32),
            pltpu.VMEM((_SUB, 128), jnp.int32),
            pltpu.VMEM((_C, _LAT), jnp.float32),
            pltpu.VMEM((_C, _LAT), jnp.float32),
            pltpu.SemaphoreType.DMA,
            pltpu.SemaphoreType.DMA,
            pltpu.SemaphoreType.DMA,
            pltpu.SemaphoreType.DMA,
            pltpu.SemaphoreType.DMA,
        ],
    )(_body)
    return k(uList, table)
